# Initial kernel scaffold; baseline (speedup 1.0000x reference)
import jax
import jax.numpy as jnp
from jax.experimental import pallas as pl


def kernel(x, edge_index, edge_type, W0, root0, b0, W1, root1, b1, fcW, fcb):
    raise NotImplementedError("write your pallas kernel here")


# speedup vs baseline: 2.5775x; 2.5775x over previous
import functools

import jax
import jax.numpy as jnp
from jax import lax
from jax.experimental import pallas as pl
from jax.experimental.pallas import tpu as pltpu
from jax.experimental.pallas import tpu_sc as plsc

NC = 2
NS = 16
L = 16

CH = 64
CC = 32
CW = 16
NR = 128
NZ = 64


def _choose_block(n, target=400):
    for b in (512, 400, 256, 250, 200, 128, 125, 100, 80, 64, 50, 40, 32, 25, 20, 16, 10, 8, 5, 4, 2, 1):
        if b <= target and n % b == 0:
            return b
    return 1


def _make_count_kernel(e_pad, cnl, half):
    cq = cnl // 8
    nch = e_pad // (CC * NS)
    NI = 4
    nnorm = cq // NS // NI
    zst = cq // NS // 32

    def body(onehot, gidx_hbm, didx0_hbm, didx1_hbm, norm_hbm,
             gidx_v, didx_v, rows_v, cbuf_v, nbuf_v, cacc, sem):
        cid = lax.axis_index("c")
        sid = lax.axis_index("s")

        for i in range(8 * NI):
            for j in range(NR // L):
                nbuf_v[i, pl.ds(j * L, L)] = jnp.zeros((L,), jnp.float32)

        for k in range(zst):
            pltpu.sync_copy(nbuf_v, cacc.at[pl.ds((sid * zst + k) * 32, 32)])
        plsc.subcore_barrier()

        @pl.loop(0, nch)
        def _(t):
            base = (t * NS + sid) * CC
            pltpu.sync_copy(gidx_hbm.at[pl.ds(base, CC)], gidx_v)

            @pl.when(cid == 0)
            def _():
                pltpu.sync_copy(didx0_hbm.at[pl.ds(base, CC)], didx_v)

            @pl.when(cid == 1)
            def _():
                pltpu.sync_copy(didx1_hbm.at[pl.ds(base, CC)], didx_v)

            pltpu.async_copy(onehot.at[gidx_v], rows_v, sem).wait()
            pltpu.sync_copy(rows_v, cacc.at[didx_v], add=True)
        plsc.subcore_barrier()

        @pl.loop(0, nnorm)
        def _(k):
            lq = (sid * nnorm + k) * NI
            pltpu.sync_copy(cacc.at[pl.ds(lq, NI)], cbuf_v)

            for i in range(NI):
                for j in range(8):
                    nv = 1.0 / jnp.maximum(cbuf_v[i, pl.ds(j * L, L)], 1.0)
                    nv = jnp.where((lq + i) * 8 + j < half, nv, 0.0)
                    for b in range(NR // L):
                        nbuf_v[i * 8 + j, pl.ds(b * L, L)] = nv

            pltpu.sync_copy(nbuf_v,
                            norm_hbm.at[pl.ds(cid * cnl + lq * 8, 8 * NI)])

    return pl.kernel(
        body,
        out_type=jax.ShapeDtypeStruct((NC * cnl, NR), jnp.float32),
        mesh=plsc.VectorSubcoreMesh(core_axis_name="c", subcore_axis_name="s"),
        scratch_types=[
            pltpu.VMEM((CC,), jnp.int32),
            pltpu.VMEM((CC,), jnp.int32),
            pltpu.VMEM((CC, NR), jnp.float32),
            pltpu.VMEM((NI, NR), jnp.float32),
            pltpu.VMEM((8 * NI, NR), jnp.float32),
            pltpu.VMEM_SHARED((cq, NR), jnp.float32),
            pltpu.SemaphoreType.DMA,
        ],
    )


def _make_norm_kernel(e_pad, cnl, half):
    nch = e_pad // (CH * NS)
    zch = cnl // NS // NZ

    def body(cidx0_hbm, cidx1_hbm, zeros_hbm, ones_hbm, norm_hbm,
             ones_v, cidx_v, cbuf_v, nbuf_v, cacc):
        cid = lax.axis_index("c")
        sid = lax.axis_index("s")

        pltpu.sync_copy(ones_hbm, ones_v)

        for k in range(zch):
            pltpu.sync_copy(zeros_hbm, cacc.at[pl.ds((sid * zch + k) * NZ, NZ)])
        plsc.subcore_barrier()

        @pl.loop(0, nch)
        def _(t):
            base = (t * NS + sid) * CH

            @pl.when(cid == 0)
            def _():
                pltpu.sync_copy(cidx0_hbm.at[pl.ds(base, CH)], cidx_v)

            @pl.when(cid == 1)
            def _():
                pltpu.sync_copy(cidx1_hbm.at[pl.ds(base, CH)], cidx_v)

            pltpu.sync_copy(ones_v, cacc.at[cidx_v], add=True)
        plsc.subcore_barrier()

        @pl.loop(0, zch)
        def _(k):
            lrow = (sid * zch + k) * NZ
            pltpu.sync_copy(cacc.at[pl.ds(lrow, NZ)], cbuf_v)

            for i in range(NZ):
                nv = 1.0 / jnp.maximum(cbuf_v[i, :], 1.0)
                nv = jnp.where(lrow + i < half, nv, 0.0)
                for j in range(NR // L):
                    nbuf_v[i, pl.ds(j * L, L)] = nv

            pltpu.sync_copy(nbuf_v, norm_hbm.at[pl.ds(cid * cnl + lrow, NZ)])

    return pl.kernel(
        body,
        out_type=jax.ShapeDtypeStruct((NC * cnl, NR), jnp.float32),
        mesh=plsc.VectorSubcoreMesh(core_axis_name="c", subcore_axis_name="s"),
        scratch_types=[
            pltpu.VMEM((CH, CW), jnp.float32),
            pltpu.VMEM((CH,), jnp.int32),
            pltpu.VMEM((NZ, CW), jnp.float32),
            pltpu.VMEM((NZ, NR), jnp.float32),
            pltpu.VMEM_SHARED((cnl, CW), jnp.float32),
        ],
    )


def _make_conv_kernel(e_pad, nh, anh, hid):
    nch = e_pad // (CC * NS)
    zch = anh // NS // NZ
    rows_main = (nh // NS) // 8 * 8
    rows_last = nh - rows_main * (NS - 1)

    def body(table, norm_hbm, gidx_hbm, cidx_hbm, didx0_hbm, didx1_hbm, part,
             gidx_v, cidx_v, didx_v, rowsn_v, rows_v, acc, sem, sem2):
        cid = lax.axis_index("c")
        sid = lax.axis_index("s")

        for e in range(CC):
            for j in range(hid // L):
                rows_v[e, pl.ds(j * L, L)] = jnp.zeros((L,), jnp.float32)

        for k in range(zch):
            for m in range(NZ // CC):
                pltpu.sync_copy(
                    rows_v,
                    acc.at[pl.ds((sid * zch + k) * NZ + m * CC, CC)])
        plsc.subcore_barrier()

        @pl.loop(0, nch)
        def _(t):
            base = (t * NS + sid) * CC
            pltpu.sync_copy(gidx_hbm.at[pl.ds(base, CC)], gidx_v)
            pltpu.sync_copy(cidx_hbm.at[pl.ds(base, CC)], cidx_v)

            @pl.when(cid == 0)
            def _():
                pltpu.sync_copy(didx0_hbm.at[pl.ds(base, CC)], didx_v)

            @pl.when(cid == 1)
            def _():
                pltpu.sync_copy(didx1_hbm.at[pl.ds(base, CC)], didx_v)

            c1 = pltpu.async_copy(table.at[gidx_v], rows_v, sem)
            c2 = pltpu.async_copy(norm_hbm.at[cidx_v], rowsn_v, sem2)
            c1.wait()
            c2.wait()

            for e in range(CC):
                sv = rowsn_v[e, pl.ds(0, L)]
                for j in range(hid // L):
                    sl = pl.ds(j * L, L)
                    rows_v[e, sl] = rows_v[e, sl] * sv

            pltpu.sync_copy(rows_v, acc.at[didx_v], add=True)

        plsc.subcore_barrier()

        @pl.when(sid < NS - 1)
        def _():
            pltpu.sync_copy(acc.at[pl.ds(sid * rows_main, rows_main)],
                            part.at[cid, pl.ds(sid * rows_main, rows_main)])

        @pl.when(sid == NS - 1)
        def _():
            pltpu.sync_copy(acc.at[pl.ds((NS - 1) * rows_main, rows_last)],
                            part.at[cid, pl.ds((NS - 1) * rows_main, rows_last)])

    return pl.kernel(
        body,
        out_type=jax.ShapeDtypeStruct((NC, nh, hid), jnp.float32),
        mesh=plsc.VectorSubcoreMesh(core_axis_name="c", subcore_axis_name="s"),
        scratch_types=[
            pltpu.VMEM((CC,), jnp.int32),
            pltpu.VMEM((CC,), jnp.int32),
            pltpu.VMEM((CC,), jnp.int32),
            pltpu.VMEM((CC, NR), jnp.float32),
            pltpu.VMEM((CC, hid), jnp.float32),
            pltpu.VMEM_SHARED((anh, hid), jnp.float32),
            pltpu.SemaphoreType.DMA,
            pltpu.SemaphoreType.DMA,
        ],
    )


def _tc_transform(x, wcat):
    n, c = x.shape
    r1, _, h = wcat.shape
    bn = _choose_block(n)

    def body(x_ref, w_ref, o_ref):
        o_ref[0] = jnp.dot(x_ref[...], w_ref[0],
                           preferred_element_type=jnp.float32)

    return pl.pallas_call(
        body,
        grid=(r1, n // bn),
        in_specs=[
            pl.BlockSpec((bn, c), lambda r, i: (i, 0)),
            pl.BlockSpec((1, c, h), lambda r, i: (r, 0, 0)),
        ],
        out_specs=pl.BlockSpec((1, bn, h), lambda r, i: (r, i, 0)),
        out_shape=jax.ShapeDtypeStruct((r1, n, h), jnp.float32),
    )(x, wcat)


def _tc_combine_transform(agg, xw_prev, b, wcat, root_slab):
    n, hid = agg.shape
    r1, _, h2 = wcat.shape
    bn = _choose_block(n)

    def body(p_ref, rt_ref, b_ref, w_ref, o_ref):
        hact = jnp.maximum(p_ref[...] + rt_ref[0] + b_ref[0], 0.0)
        for r in range(r1):
            o_ref[r] = jnp.dot(hact, w_ref[r],
                               preferred_element_type=jnp.float32)

    return pl.pallas_call(
        body,
        grid=(n // bn,),
        in_specs=[
            pl.BlockSpec((bn, hid), lambda i: (i, 0)),
            pl.BlockSpec((1, bn, hid), lambda i: (root_slab, i, 0)),
            pl.BlockSpec((1, hid), lambda i: (0, 0)),
            pl.BlockSpec((r1, hid, h2), lambda i: (0, 0, 0)),
        ],
        out_specs=pl.BlockSpec((r1, bn, h2), lambda i: (0, i, 0)),
        out_shape=jax.ShapeDtypeStruct((r1, n, h2), jnp.float32),
    )(agg, xw_prev, b, wcat)


def _tc_combine_fc(agg, xw_prev, b, fcw, fcb, root_slab):
    n, hid = agg.shape
    _, out_d = fcw.shape
    bn = _choose_block(n)

    def body(p_ref, rt_ref, b_ref, w_ref, fb_ref, o_ref):
        hact = jnp.maximum(p_ref[...] + rt_ref[0] + b_ref[0], 0.0)
        o_ref[...] = jnp.dot(hact, w_ref[...],
                             preferred_element_type=jnp.float32) + fb_ref[0]

    return pl.pallas_call(
        body,
        grid=(n // bn,),
        in_specs=[
            pl.BlockSpec((bn, hid), lambda i: (i, 0)),
            pl.BlockSpec((1, bn, hid), lambda i: (root_slab, i, 0)),
            pl.BlockSpec((1, hid), lambda i: (0, 0)),
            pl.BlockSpec((hid, out_d), lambda i: (0, 0)),
            pl.BlockSpec((1, out_d), lambda i: (0, 0)),
        ],
        out_specs=pl.BlockSpec((bn, out_d), lambda i: (i, 0)),
        out_shape=jax.ShapeDtypeStruct((n, out_d), jnp.float32),
    )(agg, xw_prev, b, fcw, fcb)


@jax.jit
def kernel(x, edge_index, edge_type, W0, root0, b0, W1, root1, b1, fcW, fcb):
    n, cin = x.shape
    r = W0.shape[0]
    hid = W0.shape[2]
    e = edge_index.shape[1]

    src = edge_index[0]
    dst = edge_index[1]
    et = edge_type

    e_pad = -(-e // (CH * NS)) * (CH * NS)
    npad = e_pad - e
    ar = jnp.arange(npad, dtype=jnp.int32)
    gidx = jnp.concatenate([et * n + src, ar % 64])

    half = r * n // 2
    cnl = -(-(half + 64) // (NS * NZ)) * (NS * NZ)
    rp = et * n + dst
    junkc = half + (rp & 63)
    padc = half + (ar % 64)
    cidx0 = jnp.concatenate([jnp.where(rp < half, rp, junkc), padc])
    cidx1 = jnp.concatenate([jnp.where(rp >= half, rp - half, junkc), padc])
    cidxg = jnp.concatenate([jnp.where(rp < half, rp, rp + (cnl - half)), padc])

    nh = n // 2
    anh = -(-(nh + 8) // (NS * NZ)) * (NS * NZ)
    junkd = nh + (dst & 7)
    padd = nh + (ar % 8)
    didx0 = jnp.concatenate([jnp.where(dst < nh, dst, junkd), padd])
    didx1 = jnp.concatenate([jnp.where(dst >= nh, dst - nh, junkd), padd])

    onehot = (jnp.arange(NR, dtype=jnp.int32)[None, :] // L
              == jnp.arange(8, dtype=jnp.int32)[:, None]).astype(jnp.float32)
    gidx_c = jnp.concatenate([rp % 8, ar % 8])
    norm = _make_count_kernel(e_pad, cnl, half)(
        onehot, gidx_c, cidx0 // 8, cidx1 // 8)

    w0cat = jnp.concatenate([W0, root0[None]], axis=0)
    w1cat = jnp.concatenate([W1, root1[None]], axis=0)

    conv = _make_conv_kernel(e_pad, nh, anh, hid)

    xw0 = _tc_transform(x, w0cat)
    part0 = conv(xw0.reshape((r + 1) * n, hid), norm, gidx, cidxg,
                 didx0, didx1)
    xw1 = _tc_combine_transform(part0.reshape(n, hid), xw0,
                                b0.reshape(1, hid), w1cat, r)
    part1 = conv(xw1.reshape((r + 1) * n, hid), norm, gidx, cidxg,
                 didx0, didx1)
    return _tc_combine_fc(part1.reshape(n, hid), xw1, b1.reshape(1, hid),
                          fcW, fcb.reshape(1, -1), r)


# speedup vs baseline: 2.9453x; 1.1427x over previous
import functools

import jax
import jax.numpy as jnp
from jax import lax
from jax.experimental import pallas as pl
from jax.experimental.pallas import tpu as pltpu
from jax.experimental.pallas import tpu_sc as plsc

NC = 2
NS = 16
L = 16

CH = 64
CC = 32
CW = 16
NR = 128
NZ = 64


def _choose_block(n, target=400):
    for b in (512, 400, 256, 250, 200, 128, 125, 100, 80, 64, 50, 40, 32, 25, 20, 16, 10, 8, 5, 4, 2, 1):
        if b <= target and n % b == 0:
            return b
    return 1


def _make_count_kernel(e_pad, cnl, half):
    cq = cnl // 8
    CHC = 64
    nch = e_pad // (CHC * NS)
    NI = 2
    nnorm = cq // NS // NI
    zst = cq // NS // (8 * NI)

    def body(onehot, pidx_hbm, norm_hbm,
             pbuf_v, rows_v, cbuf_v, nbuf_v, cacc, sem):
        cid = lax.axis_index("c")
        sid = lax.axis_index("s")

        for i in range(8 * NI):
            for j in range(NR // L):
                nbuf_v[i, pl.ds(j * L, L)] = jnp.zeros((L,), jnp.float32)

        for k in range(zst):
            pltpu.sync_copy(
                nbuf_v, cacc.at[pl.ds((sid * zst + k) * 8 * NI, 8 * NI)])
        plsc.subcore_barrier()

        @pl.loop(0, nch)
        def _(t):
            pltpu.sync_copy(pidx_hbm.at[t * NS + sid], pbuf_v)
            pltpu.async_copy(onehot.at[pbuf_v.at[0]], rows_v, sem).wait()

            @pl.when(cid == 0)
            def _():
                pltpu.sync_copy(rows_v, cacc.at[pbuf_v.at[1]], add=True)

            @pl.when(cid == 1)
            def _():
                pltpu.sync_copy(rows_v, cacc.at[pbuf_v.at[2]], add=True)
        plsc.subcore_barrier()

        @pl.loop(0, nnorm)
        def _(k):
            lq = (sid * nnorm + k) * NI
            pltpu.sync_copy(cacc.at[pl.ds(lq, NI)], cbuf_v)

            for i in range(NI):
                for j in range(8):
                    nv = 1.0 / jnp.maximum(cbuf_v[i, pl.ds(j * L, L)], 1.0)
                    nv = jnp.where((lq + i) * 8 + j < half, nv, 0.0)
                    for b in range(NR // L):
                        nbuf_v[i * 8 + j, pl.ds(b * L, L)] = nv

            pltpu.sync_copy(nbuf_v,
                            norm_hbm.at[pl.ds(cid * cnl + lq * 8, 8 * NI)])

    return pl.kernel(
        body,
        out_type=jax.ShapeDtypeStruct((NC * cnl, NR), jnp.float32),
        mesh=plsc.VectorSubcoreMesh(core_axis_name="c", subcore_axis_name="s"),
        scratch_types=[
            pltpu.VMEM((3, CHC), jnp.int32),
            pltpu.VMEM((CHC, NR), jnp.float32),
            pltpu.VMEM((NI, NR), jnp.float32),
            pltpu.VMEM((8 * NI, NR), jnp.float32),
            pltpu.VMEM_SHARED((cq, NR), jnp.float32),
            pltpu.SemaphoreType.DMA,
        ],
    )


def _make_norm_kernel(e_pad, cnl, half):
    nch = e_pad // (CH * NS)
    zch = cnl // NS // NZ

    def body(cidx0_hbm, cidx1_hbm, zeros_hbm, ones_hbm, norm_hbm,
             ones_v, cidx_v, cbuf_v, nbuf_v, cacc):
        cid = lax.axis_index("c")
        sid = lax.axis_index("s")

        pltpu.sync_copy(ones_hbm, ones_v)

        for k in range(zch):
            pltpu.sync_copy(zeros_hbm, cacc.at[pl.ds((sid * zch + k) * NZ, NZ)])
        plsc.subcore_barrier()

        @pl.loop(0, nch)
        def _(t):
            base = (t * NS + sid) * CH

            @pl.when(cid == 0)
            def _():
                pltpu.sync_copy(cidx0_hbm.at[pl.ds(base, CH)], cidx_v)

            @pl.when(cid == 1)
            def _():
                pltpu.sync_copy(cidx1_hbm.at[pl.ds(base, CH)], cidx_v)

            pltpu.sync_copy(ones_v, cacc.at[cidx_v], add=True)
        plsc.subcore_barrier()

        @pl.loop(0, zch)
        def _(k):
            lrow = (sid * zch + k) * NZ
            pltpu.sync_copy(cacc.at[pl.ds(lrow, NZ)], cbuf_v)

            for i in range(NZ):
                nv = 1.0 / jnp.maximum(cbuf_v[i, :], 1.0)
                nv = jnp.where(lrow + i < half, nv, 0.0)
                for j in range(NR // L):
                    nbuf_v[i, pl.ds(j * L, L)] = nv

            pltpu.sync_copy(nbuf_v, norm_hbm.at[pl.ds(cid * cnl + lrow, NZ)])

    return pl.kernel(
        body,
        out_type=jax.ShapeDtypeStruct((NC * cnl, NR), jnp.float32),
        mesh=plsc.VectorSubcoreMesh(core_axis_name="c", subcore_axis_name="s"),
        scratch_types=[
            pltpu.VMEM((CH, CW), jnp.float32),
            pltpu.VMEM((CH,), jnp.int32),
            pltpu.VMEM((NZ, CW), jnp.float32),
            pltpu.VMEM((NZ, NR), jnp.float32),
            pltpu.VMEM_SHARED((cnl, CW), jnp.float32),
        ],
    )


def _make_conv_kernel(e_pad, nh, anh, hid):
    nch = e_pad // (CC * NS)
    zch = anh // NS // NZ
    rows_main = (nh // NS) // 8 * 8
    rows_last = nh - rows_main * (NS - 1)

    def body(table, norm_hbm, pidx_hbm, part,
             pbuf_v, rowsn_v, rows_v, acc, sem, sem2):
        cid = lax.axis_index("c")
        sid = lax.axis_index("s")

        for e in range(CC):
            for j in range(hid // L):
                rows_v[e, pl.ds(j * L, L)] = jnp.zeros((L,), jnp.float32)

        for k in range(zch):
            for m in range(NZ // CC):
                pltpu.sync_copy(
                    rows_v,
                    acc.at[pl.ds((sid * zch + k) * NZ + m * CC, CC)])
        plsc.subcore_barrier()

        @pl.loop(0, nch)
        def _(t):
            pltpu.sync_copy(pidx_hbm.at[t * NS + sid], pbuf_v)
            c1 = pltpu.async_copy(table.at[pbuf_v.at[0]], rows_v, sem)
            c2 = pltpu.async_copy(norm_hbm.at[pbuf_v.at[1]], rowsn_v, sem2)
            c1.wait()
            c2.wait()

            for e in range(CC):
                sv = rowsn_v[e, pl.ds(0, L)]
                for j in range(hid // L):
                    sl = pl.ds(j * L, L)
                    rows_v[e, sl] = rows_v[e, sl] * sv

            @pl.when(cid == 0)
            def _():
                pltpu.sync_copy(rows_v, acc.at[pbuf_v.at[2]], add=True)

            @pl.when(cid == 1)
            def _():
                pltpu.sync_copy(rows_v, acc.at[pbuf_v.at[3]], add=True)

        plsc.subcore_barrier()

        @pl.when(sid < NS - 1)
        def _():
            pltpu.sync_copy(acc.at[pl.ds(sid * rows_main, rows_main)],
                            part.at[cid, pl.ds(sid * rows_main, rows_main)])

        @pl.when(sid == NS - 1)
        def _():
            pltpu.sync_copy(acc.at[pl.ds((NS - 1) * rows_main, rows_last)],
                            part.at[cid, pl.ds((NS - 1) * rows_main, rows_last)])

    return pl.kernel(
        body,
        out_type=jax.ShapeDtypeStruct((NC, nh, hid), jnp.float32),
        mesh=plsc.VectorSubcoreMesh(core_axis_name="c", subcore_axis_name="s"),
        scratch_types=[
            pltpu.VMEM((4, CC), jnp.int32),
            pltpu.VMEM((CC, NR), jnp.float32),
            pltpu.VMEM((CC, hid), jnp.float32),
            pltpu.VMEM_SHARED((anh, hid), jnp.float32),
            pltpu.SemaphoreType.DMA,
            pltpu.SemaphoreType.DMA,
        ],
    )


def _tc_transform(x, wcat):
    n, c = x.shape
    r1, _, h = wcat.shape
    bn = _choose_block(n)

    def body(x_ref, w_ref, o_ref):
        o_ref[0] = jnp.dot(x_ref[...], w_ref[0],
                           preferred_element_type=jnp.float32)

    return pl.pallas_call(
        body,
        grid=(r1, n // bn),
        in_specs=[
            pl.BlockSpec((bn, c), lambda r, i: (i, 0)),
            pl.BlockSpec((1, c, h), lambda r, i: (r, 0, 0)),
        ],
        out_specs=pl.BlockSpec((1, bn, h), lambda r, i: (r, i, 0)),
        out_shape=jax.ShapeDtypeStruct((r1, n, h), jnp.float32),
    )(x, wcat)


def _tc_combine_transform(agg, xw_prev, b, wcat, root_slab):
    n, hid = agg.shape
    r1, _, h2 = wcat.shape
    bn = _choose_block(n)

    def body(p_ref, rt_ref, b_ref, w_ref, o_ref):
        hact = jnp.maximum(p_ref[...] + rt_ref[0] + b_ref[0], 0.0)
        for r in range(r1):
            o_ref[r] = jnp.dot(hact, w_ref[r],
                               preferred_element_type=jnp.float32)

    return pl.pallas_call(
        body,
        grid=(n // bn,),
        in_specs=[
            pl.BlockSpec((bn, hid), lambda i: (i, 0)),
            pl.BlockSpec((1, bn, hid), lambda i: (root_slab, i, 0)),
            pl.BlockSpec((1, hid), lambda i: (0, 0)),
            pl.BlockSpec((r1, hid, h2), lambda i: (0, 0, 0)),
        ],
        out_specs=pl.BlockSpec((r1, bn, h2), lambda i: (0, i, 0)),
        out_shape=jax.ShapeDtypeStruct((r1, n, h2), jnp.float32),
    )(agg, xw_prev, b, wcat)


def _tc_combine_fc(agg, xw_prev, b, fcw, fcb, root_slab):
    n, hid = agg.shape
    _, out_d = fcw.shape
    bn = _choose_block(n)

    def body(p_ref, rt_ref, b_ref, w_ref, fb_ref, o_ref):
        hact = jnp.maximum(p_ref[...] + rt_ref[0] + b_ref[0], 0.0)
        o_ref[...] = jnp.dot(hact, w_ref[...],
                             preferred_element_type=jnp.float32) + fb_ref[0]

    return pl.pallas_call(
        body,
        grid=(n // bn,),
        in_specs=[
            pl.BlockSpec((bn, hid), lambda i: (i, 0)),
            pl.BlockSpec((1, bn, hid), lambda i: (root_slab, i, 0)),
            pl.BlockSpec((1, hid), lambda i: (0, 0)),
            pl.BlockSpec((hid, out_d), lambda i: (0, 0)),
            pl.BlockSpec((1, out_d), lambda i: (0, 0)),
        ],
        out_specs=pl.BlockSpec((bn, out_d), lambda i: (i, 0)),
        out_shape=jax.ShapeDtypeStruct((n, out_d), jnp.float32),
    )(agg, xw_prev, b, fcw, fcb)


@jax.jit
def kernel(x, edge_index, edge_type, W0, root0, b0, W1, root1, b1, fcW, fcb):
    n, cin = x.shape
    r = W0.shape[0]
    hid = W0.shape[2]
    e = edge_index.shape[1]

    src = edge_index[0]
    dst = edge_index[1]
    et = edge_type

    e_pad = -(-e // (CH * NS)) * (CH * NS)
    npad = e_pad - e
    ar = jnp.arange(npad, dtype=jnp.int32)
    gidx = jnp.concatenate([et * n + src, ar % 64])

    half = r * n // 2
    cnl = -(-(half + 64) // (NS * NZ)) * (NS * NZ)
    rp = et * n + dst
    junkc = half + (rp & 63)
    padc = half + (ar % 64)
    cidx0 = jnp.concatenate([jnp.where(rp < half, rp, junkc), padc])
    cidx1 = jnp.concatenate([jnp.where(rp >= half, rp - half, junkc), padc])
    cidxg = jnp.concatenate([jnp.where(rp < half, rp, rp + (cnl - half)), padc])

    nh = n // 2
    anh = -(-(nh + 8) // (NS * NZ)) * (NS * NZ)
    junkd = nh + (dst & 7)
    padd = nh + (ar % 8)
    didx0 = jnp.concatenate([jnp.where(dst < nh, dst, junkd), padd])
    didx1 = jnp.concatenate([jnp.where(dst >= nh, dst - nh, junkd), padd])

    onehot = (jnp.arange(NR, dtype=jnp.int32)[None, :] // L
              == jnp.arange(8, dtype=jnp.int32)[:, None]).astype(jnp.float32)
    gidx_c = jnp.concatenate([rp % 8, ar % 8])
    pidx_c = jnp.stack([gidx_c, cidx0 // 8, cidx1 // 8]).reshape(
        3, e_pad // 64, 64).transpose(1, 0, 2)
    norm = _make_count_kernel(e_pad, cnl, half)(onehot, pidx_c)
    pidx = jnp.stack([gidx, cidxg, didx0, didx1]).reshape(
        4, e_pad // CC, CC).transpose(1, 0, 2)

    w0cat = jnp.concatenate([W0, root0[None]], axis=0)
    w1cat = jnp.concatenate([W1, root1[None]], axis=0)

    conv = _make_conv_kernel(e_pad, nh, anh, hid)

    xw0 = _tc_transform(x, w0cat)
    part0 = conv(xw0.reshape((r + 1) * n, hid), norm, pidx)
    xw1 = _tc_combine_transform(part0.reshape(n, hid), xw0,
                                b0.reshape(1, hid), w1cat, r)
    part1 = conv(xw1.reshape((r + 1) * n, hid), norm, pidx)
    return _tc_combine_fc(part1.reshape(n, hid), xw1, b1.reshape(1, hid),
                          fcW, fcb.reshape(1, -1), r)
